# baseline (speedup 1.0000x reference)
import jax
import jax.numpy as jnp
from jax.experimental import pallas as pl

_R = 10000


def _body(an_ref, tab_ref, x_ref, o_ref):
    an_row = an_ref[...].reshape(1, _R)
    rs = jax.lax.rsqrt(tab_ref[...])
    sub = jax.lax.broadcasted_iota(jnp.int32, (128, _R), 0)
    onehot_t = (sub == an_row).astype(jnp.float32)
    rsmat = jnp.broadcast_to(
        rs.reshape(128, 1), (128, 128)
    )
    s_b = jax.lax.dot_general(
        onehot_t, rsmat,
        dimension_numbers=(((0,), (0,)), ((), ())),
        preferred_element_type=jnp.float32,
    )
    o_ref[...] = x_ref[...] * s_b


def kernel(x, atomic_numbers, scales):
    n, d = x.shape
    nb = n // _R
    an = atomic_numbers.astype(jnp.int32).reshape(nb, 1, _R)
    tab = jnp.concatenate(
        [scales[:, 0], jnp.ones((128 - scales.shape[0],), jnp.float32)]
    ).reshape(1, 128)
    return pl.pallas_call(
        _body,
        grid=(nb,),
        in_specs=[
            pl.BlockSpec((1, 1, _R), lambda i: (i, 0, 0)),
            pl.BlockSpec((1, 128), lambda i: (0, 0)),
            pl.BlockSpec((_R, d), lambda i: (i, 0)),
        ],
        out_specs=pl.BlockSpec((_R, d), lambda i: (i, 0)),
        out_shape=jax.ShapeDtypeStruct((n, d), x.dtype),
    )(an, tab, x)


# speedup vs baseline: 1.0295x; 1.0295x over previous
import jax
import jax.numpy as jnp
from jax.experimental import pallas as pl

_R = 20000


def _body(an_ref, tab_ref, x_ref, o_ref):
    an_row = an_ref[...].reshape(1, _R)
    rs = jax.lax.rsqrt(tab_ref[...])
    sub = jax.lax.broadcasted_iota(jnp.int32, (128, _R), 0)
    onehot_t = (sub == an_row).astype(jnp.bfloat16)
    rsmat = jnp.broadcast_to(
        rs.reshape(128, 1), (128, 128)
    ).astype(jnp.bfloat16)
    s_b = jax.lax.dot_general(
        onehot_t, rsmat,
        dimension_numbers=(((0,), (0,)), ((), ())),
        preferred_element_type=jnp.float32,
    )
    o_ref[...] = x_ref[...] * s_b


def kernel(x, atomic_numbers, scales):
    n, d = x.shape
    nb = n // _R
    an = atomic_numbers.astype(jnp.int32).reshape(nb, 1, _R)
    tab = jnp.concatenate(
        [scales[:, 0], jnp.ones((128 - scales.shape[0],), jnp.float32)]
    ).reshape(1, 128)
    return pl.pallas_call(
        _body,
        grid=(nb,),
        in_specs=[
            pl.BlockSpec((1, 1, _R), lambda i: (i, 0, 0)),
            pl.BlockSpec((1, 128), lambda i: (0, 0)),
            pl.BlockSpec((_R, d), lambda i: (i, 0)),
        ],
        out_specs=pl.BlockSpec((_R, d), lambda i: (i, 0)),
        out_shape=jax.ShapeDtypeStruct((n, d), x.dtype),
    )(an, tab, x)
